# baseline (speedup 1.0000x reference)
import functools

import jax
import jax.numpy as jnp
from jax import lax
from jax.experimental import pallas as pl
from jax.experimental.pallas import tpu as pltpu
from jax.experimental.pallas import tpu_sc as plsc

_NUM_SLOTS = 64
_MIN_K = 2
_MAX_K = 8
_NEG_BIG = -1000000000.0
_N_WORKERS = 32
_TOK_PER_W = 256


def _score_body(r_dim, x_ref, wcat_ref, ws_ref, bh_ref, bs_ref, bc_ref,
                scores_ref, budget_ref, probs_ref):
    i = pl.program_id(0)

    xb = x_ref[...].astype(jnp.bfloat16)
    hfull = jnp.dot(xb, wcat_ref[...], preferred_element_type=jnp.float32)
    h = jnp.maximum(hfull[:, :r_dim] + bh_ref[...], 0.0)
    sc = jnp.dot(h.astype(jnp.bfloat16), ws_ref[...],
                 preferred_element_type=jnp.float32) + bs_ref[...]

    c = hfull[:, r_dim:r_dim + 1] + bc_ref[0]
    s = jax.nn.sigmoid(c)
    budget = jnp.floor(_MIN_K + (_MAX_K - _MIN_K) * s * s).astype(jnp.int32)
    budget_ref[...] = budget

    scores_ref[...] = sc.T

    pe = jnp.exp(sc - jnp.max(sc, axis=1, keepdims=True))
    probs = pe / jnp.sum(pe, axis=1, keepdims=True)

    @pl.when(i == 0)
    def _init():
        probs_ref[...] = jnp.zeros_like(probs_ref)

    probs_ref[...] += jnp.sum(probs, axis=0, keepdims=True)


def _sc_route_body(tokw, scores_hbm, budget_hbm, ow_hbm, oi_hbm, cnt_hbm,
                   work, bv, ow, oi, cnt):
    wid = lax.axis_index("s") * 2 + lax.axis_index("c")
    base = wid * tokw
    pltpu.sync_copy(scores_hbm.at[:, pl.ds(base, tokw)], work)
    pltpu.sync_copy(budget_hbm.at[pl.ds(base, tokw)], bv)

    z16 = jnp.zeros((16,), jnp.float32)
    for j in range(_NUM_SLOTS // 16):
        cnt[pl.ds(j * 16, 16)] = z16

    iota16 = lax.iota(jnp.int32, 16)
    ones16 = jnp.ones((16,), jnp.float32)
    neg16 = jnp.full((16,), -1.0e30, jnp.float32)

    def group(g, carry):
        col = g * 16
        rows = iota16 + col
        b = bv[pl.ds(col, 16)]

        vals = []
        idxs = []
        for k in range(_MAX_K):
            m = jnp.full((16,), -3.0e38, jnp.float32)
            am = jnp.zeros((16,), jnp.int32)
            for slot in range(_NUM_SLOTS):
                v = work[slot, pl.ds(col, 16)]
                gt = v > m
                m = jnp.where(gt, v, m)
                am = jnp.where(gt, slot, am)
            vals.append(m)
            idxs.append(am)
            plsc.store_scatter(work, [am, rows], neg16)

        mx = vals[0]
        denom = jnp.zeros((16,), jnp.float32)
        es = []
        for k in range(_MAX_K):
            mk = b > k
            ml = jnp.where(mk, vals[k], jnp.float32(_NEG_BIG))
            e = jnp.exp(ml - mx)
            es.append((e, mk))
            denom = denom + e
        for k in range(_MAX_K):
            e, mk = es[k]
            wk = jnp.where(mk, e / denom, jnp.float32(0.0))
            ow[k, pl.ds(col, 16)] = wk
            oi[k, pl.ds(col, 16)] = idxs[k]
            plsc.addupdate_scatter(cnt, [idxs[k]], ones16, mask=mk)
        return carry

    lax.fori_loop(0, tokw // 16, group, 0)

    pltpu.sync_copy(ow, ow_hbm.at[:, pl.ds(base, tokw)])
    pltpu.sync_copy(oi, oi_hbm.at[:, pl.ds(base, tokw)])
    pltpu.sync_copy(cnt, cnt_hbm.at[wid])


def _aux_body(n_tokens, cnt_ref, probs_ref, aux_ref):
    counts = jnp.sum(cnt_ref[...], axis=0, keepdims=True)
    probs = jnp.sum(probs_ref[...], axis=0, keepdims=True)
    total = jnp.float32(n_tokens)
    aux = _NUM_SLOTS * jnp.sum((probs / total) * (counts / total))
    aux_ref[...] = jnp.full((1, 1), aux, jnp.float32)


def kernel(x, W_c, b_c, W_h, b_h, W_s, b_s):
    B, S, D = x.shape
    N = B * S
    R = W_h.shape[1]
    TB = 1024
    grid = N // TB

    x2 = x.reshape(N, D)
    ncat = R + 128
    wcat = jnp.concatenate(
        [W_h, W_c, jnp.zeros((D, ncat - R - 1), W_h.dtype)],
        axis=1).astype(jnp.bfloat16)
    wsb = W_s.astype(jnp.bfloat16)
    bh2 = b_h.reshape(1, R)
    bs2 = b_s.reshape(1, _NUM_SLOTS)

    CH = 2
    Nc = N // CH
    tokw = Nc // _N_WORKERS
    gridc = Nc // TB

    mesh = plsc.VectorSubcoreMesh(core_axis_name="c", subcore_axis_name="s")
    sc_route = functools.partial(
        pl.kernel, mesh=mesh,
        out_type=[
            jax.ShapeDtypeStruct((_MAX_K, Nc), jnp.float32),
            jax.ShapeDtypeStruct((_MAX_K, Nc), jnp.int32),
            jax.ShapeDtypeStruct((_N_WORKERS, _NUM_SLOTS), jnp.float32),
        ],
        scratch_types=[
            pltpu.VMEM((_NUM_SLOTS, tokw), jnp.float32),
            pltpu.VMEM((tokw,), jnp.int32),
            pltpu.VMEM((_MAX_K, tokw), jnp.float32),
            pltpu.VMEM((_MAX_K, tokw), jnp.int32),
            pltpu.VMEM((_NUM_SLOTS,), jnp.float32),
        ],
        compiler_params=pltpu.CompilerParams(
            use_tc_tiling_on_sc=False, needs_layout_passes=False),
    )(functools.partial(_sc_route_body, tokw))

    ows, ois, cnts, budgets, probss = [], [], [], [], []
    for ci in range(CH):
        scores, budget, probs = pl.pallas_call(
            functools.partial(_score_body, R),
            grid=(gridc,),
            in_specs=[
                pl.BlockSpec((TB, D), lambda i, ci=ci: (ci * gridc + i, 0)),
                pl.BlockSpec((D, ncat), lambda i: (0, 0)),
                pl.BlockSpec((R, _NUM_SLOTS), lambda i: (0, 0)),
                pl.BlockSpec((1, R), lambda i: (0, 0)),
                pl.BlockSpec((1, _NUM_SLOTS), lambda i: (0, 0)),
                pl.BlockSpec(memory_space=pltpu.SMEM),
            ],
            out_specs=[
                pl.BlockSpec((_NUM_SLOTS, TB), lambda i: (0, i)),
                pl.BlockSpec((TB, 1), lambda i: (i, 0)),
                pl.BlockSpec((1, _NUM_SLOTS), lambda i: (0, 0)),
            ],
            out_shape=[
                jax.ShapeDtypeStruct((_NUM_SLOTS, Nc), jnp.float32),
                jax.ShapeDtypeStruct((Nc, 1), jnp.int32),
                jax.ShapeDtypeStruct((1, _NUM_SLOTS), jnp.float32),
            ],
            compiler_params=pltpu.CompilerParams(
                dimension_semantics=("arbitrary",),
            ),
        )(x2, wcat, wsb, bh2, bs2, b_c)
        ow, oi, cnt_rows = sc_route(scores, budget.reshape(Nc))
        ows.append(ow)
        ois.append(oi)
        cnts.append(cnt_rows)
        budgets.append(budget)
        probss.append(probs)

    aux = pl.pallas_call(
        functools.partial(_aux_body, N),
        out_shape=jax.ShapeDtypeStruct((1, 1), jnp.float32),
    )(jnp.concatenate(cnts, axis=0), jnp.concatenate(probss, axis=0))

    oi_full = jnp.concatenate(ois, axis=1)
    ow_full = jnp.concatenate(ows, axis=1)
    budget_full = jnp.concatenate(budgets, axis=0)
    return (oi_full.T.reshape(B, S, _MAX_K), ow_full.T.reshape(B, S, _MAX_K),
            budget_full.reshape(B, S, 1), aux[0, 0])


# speedup vs baseline: 1.0393x; 1.0393x over previous
import functools

import jax
import jax.numpy as jnp
from jax import lax
from jax.experimental import pallas as pl
from jax.experimental.pallas import tpu as pltpu
from jax.experimental.pallas import tpu_sc as plsc

_NUM_SLOTS = 64
_MIN_K = 2
_MAX_K = 8
_NEG_BIG = -1000000000.0
_N_WORKERS = 32
_TOK_PER_W = 256


def _score_body(r_dim, x_ref, wcat_ref, ws_ref, bh_ref, bs_ref, bc_ref,
                scores_ref, budget_ref, probs_ref):
    i = pl.program_id(0)

    xb = x_ref[...].astype(jnp.bfloat16)
    hfull = jnp.dot(xb, wcat_ref[...], preferred_element_type=jnp.float32)
    h = jnp.maximum(hfull[:, :r_dim] + bh_ref[...], 0.0)
    sc = jnp.dot(h.astype(jnp.bfloat16), ws_ref[...],
                 preferred_element_type=jnp.float32) + bs_ref[...]

    c = hfull[:, r_dim:r_dim + 1] + bc_ref[0]
    s = jax.nn.sigmoid(c)
    budget = jnp.floor(_MIN_K + (_MAX_K - _MIN_K) * s * s).astype(jnp.int32)
    budget_ref[...] = budget

    scores_ref[...] = sc.T

    pe = jnp.exp(sc - jnp.max(sc, axis=1, keepdims=True))
    probs = pe / jnp.sum(pe, axis=1, keepdims=True)

    @pl.when(i == 0)
    def _init():
        probs_ref[...] = jnp.zeros_like(probs_ref)

    probs_ref[...] += jnp.sum(probs, axis=0, keepdims=True)


def _sc_route_body(tokw, scores_hbm, budget_hbm, ow_hbm, oi_hbm, cnt_hbm,
                   work, bv, ow, oi, cnt):
    wid = lax.axis_index("s") * 2 + lax.axis_index("c")
    base = wid * tokw
    pltpu.sync_copy(scores_hbm.at[:, pl.ds(base, tokw)], work)
    pltpu.sync_copy(budget_hbm.at[pl.ds(base, tokw)], bv)

    z16 = jnp.zeros((16,), jnp.float32)
    for j in range(_NUM_SLOTS // 16):
        cnt[pl.ds(j * 16, 16)] = z16

    iota16 = lax.iota(jnp.int32, 16)
    ones16 = jnp.ones((16,), jnp.float32)
    neg16 = jnp.full((16,), -1.0e30, jnp.float32)

    def group(g, carry):
        col = g * 16
        rows = iota16 + col
        b = bv[pl.ds(col, 16)]

        vals = []
        idxs = []
        for k in range(_MAX_K):
            m = jnp.full((16,), -3.0e38, jnp.float32)
            am = jnp.zeros((16,), jnp.int32)
            for slot in range(_NUM_SLOTS):
                v = work[slot, pl.ds(col, 16)]
                gt = v > m
                m = jnp.where(gt, v, m)
                am = jnp.where(gt, slot, am)
            vals.append(m)
            idxs.append(am)
            plsc.store_scatter(work, [am, rows], neg16)

        mx = vals[0]
        denom = jnp.zeros((16,), jnp.float32)
        es = []
        for k in range(_MAX_K):
            mk = b > k
            ml = jnp.where(mk, vals[k], jnp.float32(_NEG_BIG))
            e = jnp.exp(ml - mx)
            es.append((e, mk))
            denom = denom + e
        for k in range(_MAX_K):
            e, mk = es[k]
            wk = jnp.where(mk, e / denom, jnp.float32(0.0))
            ow[k, pl.ds(col, 16)] = wk
            oi[k, pl.ds(col, 16)] = idxs[k]
            plsc.addupdate_scatter(cnt, [idxs[k]], ones16, mask=mk)
        return carry

    lax.fori_loop(0, tokw // 16, group, 0)

    pltpu.sync_copy(ow, ow_hbm.at[:, pl.ds(base, tokw)])
    pltpu.sync_copy(oi, oi_hbm.at[:, pl.ds(base, tokw)])
    pltpu.sync_copy(cnt, cnt_hbm.at[wid])


def _aux_body(n_tokens, cnt_ref, probs_ref, aux_ref):
    counts = jnp.sum(cnt_ref[...], axis=0, keepdims=True)
    probs = jnp.sum(probs_ref[...], axis=0, keepdims=True)
    total = jnp.float32(n_tokens)
    aux = _NUM_SLOTS * jnp.sum((probs / total) * (counts / total))
    aux_ref[...] = jnp.full((1, 1), aux, jnp.float32)


def kernel(x, W_c, b_c, W_h, b_h, W_s, b_s):
    B, S, D = x.shape
    N = B * S
    R = W_h.shape[1]
    TB = 2048

    x2 = x.reshape(N, D)
    ncat = R + 128
    wcat = jnp.concatenate(
        [W_h, W_c, jnp.zeros((D, ncat - R - 1), W_h.dtype)],
        axis=1).astype(jnp.bfloat16)
    wsb = W_s.astype(jnp.bfloat16)
    bh2 = b_h.reshape(1, R)
    bs2 = b_s.reshape(1, _NUM_SLOTS)

    CH = 1
    Nc = N // CH
    tokw = Nc // _N_WORKERS
    gridc = Nc // TB

    mesh = plsc.VectorSubcoreMesh(core_axis_name="c", subcore_axis_name="s")
    sc_route = functools.partial(
        pl.kernel, mesh=mesh,
        out_type=[
            jax.ShapeDtypeStruct((_MAX_K, Nc), jnp.float32),
            jax.ShapeDtypeStruct((_MAX_K, Nc), jnp.int32),
            jax.ShapeDtypeStruct((_N_WORKERS, _NUM_SLOTS), jnp.float32),
        ],
        scratch_types=[
            pltpu.VMEM((_NUM_SLOTS, tokw), jnp.float32),
            pltpu.VMEM((tokw,), jnp.int32),
            pltpu.VMEM((_MAX_K, tokw), jnp.float32),
            pltpu.VMEM((_MAX_K, tokw), jnp.int32),
            pltpu.VMEM((_NUM_SLOTS,), jnp.float32),
        ],
        compiler_params=pltpu.CompilerParams(
            use_tc_tiling_on_sc=False, needs_layout_passes=False),
    )(functools.partial(_sc_route_body, tokw))

    ows, ois, cnts, budgets, probss = [], [], [], [], []
    for ci in range(CH):
        scores, budget, probs = pl.pallas_call(
            functools.partial(_score_body, R),
            grid=(gridc,),
            in_specs=[
                pl.BlockSpec((TB, D), lambda i, ci=ci: (ci * gridc + i, 0)),
                pl.BlockSpec((D, ncat), lambda i: (0, 0)),
                pl.BlockSpec((R, _NUM_SLOTS), lambda i: (0, 0)),
                pl.BlockSpec((1, R), lambda i: (0, 0)),
                pl.BlockSpec((1, _NUM_SLOTS), lambda i: (0, 0)),
                pl.BlockSpec(memory_space=pltpu.SMEM),
            ],
            out_specs=[
                pl.BlockSpec((_NUM_SLOTS, TB), lambda i: (0, i)),
                pl.BlockSpec((TB, 1), lambda i: (i, 0)),
                pl.BlockSpec((1, _NUM_SLOTS), lambda i: (0, 0)),
            ],
            out_shape=[
                jax.ShapeDtypeStruct((_NUM_SLOTS, Nc), jnp.float32),
                jax.ShapeDtypeStruct((Nc, 1), jnp.int32),
                jax.ShapeDtypeStruct((1, _NUM_SLOTS), jnp.float32),
            ],
            compiler_params=pltpu.CompilerParams(
                dimension_semantics=("arbitrary",),
            ),
        )(x2, wcat, wsb, bh2, bs2, b_c)
        ow, oi, cnt_rows = sc_route(scores, budget.reshape(Nc))
        ows.append(ow)
        ois.append(oi)
        cnts.append(cnt_rows)
        budgets.append(budget)
        probss.append(probs)

    aux = pl.pallas_call(
        functools.partial(_aux_body, N),
        out_shape=jax.ShapeDtypeStruct((1, 1), jnp.float32),
    )(jnp.concatenate(cnts, axis=0), jnp.concatenate(probss, axis=0))

    oi_full = jnp.concatenate(ois, axis=1)
    ow_full = jnp.concatenate(ows, axis=1)
    budget_full = jnp.concatenate(budgets, axis=0)
    return (oi_full.T.reshape(B, S, _MAX_K), ow_full.T.reshape(B, S, _MAX_K),
            budget_full.reshape(B, S, 1), aux[0, 0])


# speedup vs baseline: 1.0955x; 1.0541x over previous
import functools

import jax
import jax.numpy as jnp
from jax import lax
from jax.experimental import pallas as pl
from jax.experimental.pallas import tpu as pltpu
from jax.experimental.pallas import tpu_sc as plsc

_NUM_SLOTS = 64
_MIN_K = 2
_MAX_K = 8
_NEG_BIG = -1000000000.0
_N_WORKERS = 32
_TOK_PER_W = 256


def _score_body(r_dim, x_ref, wcat_ref, ws_ref, bh_ref, bs_ref, bc_ref,
                scores_ref, budget_ref, probs_ref):
    i = pl.program_id(0)

    xb = x_ref[...].astype(jnp.bfloat16)
    hfull = jnp.dot(xb, wcat_ref[...], preferred_element_type=jnp.float32)
    h = jnp.maximum(hfull[:, :r_dim] + bh_ref[...], 0.0)
    sc = jnp.dot(h.astype(jnp.bfloat16), ws_ref[...],
                 preferred_element_type=jnp.float32) + bs_ref[...]

    c = hfull[:, r_dim:r_dim + 1] + bc_ref[0]
    s = jax.nn.sigmoid(c)
    budget = jnp.floor(_MIN_K + (_MAX_K - _MIN_K) * s * s).astype(jnp.int32)
    budget_ref[...] = budget

    scores_ref[...] = sc.T

    pe = jnp.exp(sc - jnp.max(sc, axis=1, keepdims=True))
    probs = pe / jnp.sum(pe, axis=1, keepdims=True)

    @pl.when(i == 0)
    def _init():
        probs_ref[...] = jnp.zeros_like(probs_ref)

    probs_ref[...] += jnp.sum(probs, axis=0, keepdims=True)


def _sc_route_body(tokw, scores_hbm, budget_hbm, ow_hbm, oi_hbm, cnt_hbm,
                   work, bv, ow, oi, cnt):
    wid = lax.axis_index("s") * 2 + lax.axis_index("c")
    base = wid * tokw
    pltpu.sync_copy(scores_hbm.at[:, pl.ds(base, tokw)], work)
    pltpu.sync_copy(budget_hbm.at[pl.ds(base, tokw)], bv)

    z16 = jnp.zeros((16,), jnp.float32)
    for j in range(_NUM_SLOTS // 16):
        cnt[pl.ds(j * 16, 16)] = z16

    iota16 = lax.iota(jnp.int32, 16)
    ones16 = jnp.ones((16,), jnp.float32)
    neg16 = jnp.full((16,), -1.0e30, jnp.float32)

    def group(g, carry):
        col = g * 16
        rows = iota16 + col
        b = bv[pl.ds(col, 16)]

        vals = []
        idxs = []
        for k in range(_MAX_K):
            ms = []
            ams = []
            for h in range(4):
                m = jnp.full((16,), -3.0e38, jnp.float32)
                am = jnp.zeros((16,), jnp.int32)
                for slot in range(h * 16, (h + 1) * 16):
                    v = work[slot, pl.ds(col, 16)]
                    gt = v > m
                    m = jnp.where(gt, v, m)
                    am = jnp.where(gt, slot, am)
                ms.append(m)
                ams.append(am)
            g01 = ms[1] > ms[0]
            m01 = jnp.where(g01, ms[1], ms[0])
            a01 = jnp.where(g01, ams[1], ams[0])
            g23 = ms[3] > ms[2]
            m23 = jnp.where(g23, ms[3], ms[2])
            a23 = jnp.where(g23, ams[3], ams[2])
            gf = m23 > m01
            m = jnp.where(gf, m23, m01)
            am = jnp.where(gf, a23, a01)
            vals.append(m)
            idxs.append(am)
            plsc.store_scatter(work, [am, rows], neg16)

        mx = vals[0]
        denom = jnp.zeros((16,), jnp.float32)
        es = []
        for k in range(_MAX_K):
            mk = b > k
            ml = jnp.where(mk, vals[k], jnp.float32(_NEG_BIG))
            e = jnp.exp(ml - mx)
            es.append((e, mk))
            denom = denom + e
        for k in range(_MAX_K):
            e, mk = es[k]
            wk = jnp.where(mk, e / denom, jnp.float32(0.0))
            ow[k, pl.ds(col, 16)] = wk
            oi[k, pl.ds(col, 16)] = idxs[k]
            plsc.addupdate_scatter(cnt, [idxs[k]], ones16, mask=mk)
        return carry

    lax.fori_loop(0, tokw // 16, group, 0)

    pltpu.sync_copy(ow, ow_hbm.at[:, pl.ds(base, tokw)])
    pltpu.sync_copy(oi, oi_hbm.at[:, pl.ds(base, tokw)])
    pltpu.sync_copy(cnt, cnt_hbm.at[wid])


def _aux_body(n_tokens, cnt_ref, probs_ref, aux_ref):
    counts = jnp.sum(cnt_ref[...], axis=0, keepdims=True)
    probs = jnp.sum(probs_ref[...], axis=0, keepdims=True)
    total = jnp.float32(n_tokens)
    aux = _NUM_SLOTS * jnp.sum((probs / total) * (counts / total))
    aux_ref[...] = jnp.full((1, 1), aux, jnp.float32)


def kernel(x, W_c, b_c, W_h, b_h, W_s, b_s):
    B, S, D = x.shape
    N = B * S
    R = W_h.shape[1]
    TB = 2048

    x2 = x.reshape(N, D)
    ncat = R + 128
    wcat = jnp.concatenate(
        [W_h, W_c, jnp.zeros((D, ncat - R - 1), W_h.dtype)],
        axis=1).astype(jnp.bfloat16)
    wsb = W_s.astype(jnp.bfloat16)
    bh2 = b_h.reshape(1, R)
    bs2 = b_s.reshape(1, _NUM_SLOTS)

    CH = 1
    Nc = N // CH
    tokw = Nc // _N_WORKERS
    gridc = Nc // TB

    mesh = plsc.VectorSubcoreMesh(core_axis_name="c", subcore_axis_name="s")
    sc_route = functools.partial(
        pl.kernel, mesh=mesh,
        out_type=[
            jax.ShapeDtypeStruct((_MAX_K, Nc), jnp.float32),
            jax.ShapeDtypeStruct((_MAX_K, Nc), jnp.int32),
            jax.ShapeDtypeStruct((_N_WORKERS, _NUM_SLOTS), jnp.float32),
        ],
        scratch_types=[
            pltpu.VMEM((_NUM_SLOTS, tokw), jnp.float32),
            pltpu.VMEM((tokw,), jnp.int32),
            pltpu.VMEM((_MAX_K, tokw), jnp.float32),
            pltpu.VMEM((_MAX_K, tokw), jnp.int32),
            pltpu.VMEM((_NUM_SLOTS,), jnp.float32),
        ],
        compiler_params=pltpu.CompilerParams(
            use_tc_tiling_on_sc=False, needs_layout_passes=False),
    )(functools.partial(_sc_route_body, tokw))

    ows, ois, cnts, budgets, probss = [], [], [], [], []
    for ci in range(CH):
        scores, budget, probs = pl.pallas_call(
            functools.partial(_score_body, R),
            grid=(gridc,),
            in_specs=[
                pl.BlockSpec((TB, D), lambda i, ci=ci: (ci * gridc + i, 0)),
                pl.BlockSpec((D, ncat), lambda i: (0, 0)),
                pl.BlockSpec((R, _NUM_SLOTS), lambda i: (0, 0)),
                pl.BlockSpec((1, R), lambda i: (0, 0)),
                pl.BlockSpec((1, _NUM_SLOTS), lambda i: (0, 0)),
                pl.BlockSpec(memory_space=pltpu.SMEM),
            ],
            out_specs=[
                pl.BlockSpec((_NUM_SLOTS, TB), lambda i: (0, i)),
                pl.BlockSpec((TB, 1), lambda i: (i, 0)),
                pl.BlockSpec((1, _NUM_SLOTS), lambda i: (0, 0)),
            ],
            out_shape=[
                jax.ShapeDtypeStruct((_NUM_SLOTS, Nc), jnp.float32),
                jax.ShapeDtypeStruct((Nc, 1), jnp.int32),
                jax.ShapeDtypeStruct((1, _NUM_SLOTS), jnp.float32),
            ],
            compiler_params=pltpu.CompilerParams(
                dimension_semantics=("arbitrary",),
            ),
        )(x2, wcat, wsb, bh2, bs2, b_c)
        ow, oi, cnt_rows = sc_route(scores, budget.reshape(Nc))
        ows.append(ow)
        ois.append(oi)
        cnts.append(cnt_rows)
        budgets.append(budget)
        probss.append(probs)

    aux = pl.pallas_call(
        functools.partial(_aux_body, N),
        out_shape=jax.ShapeDtypeStruct((1, 1), jnp.float32),
    )(jnp.concatenate(cnts, axis=0), jnp.concatenate(probss, axis=0))

    oi_full = jnp.concatenate(ois, axis=1)
    ow_full = jnp.concatenate(ows, axis=1)
    budget_full = jnp.concatenate(budgets, axis=0)
    return (oi_full.T.reshape(B, S, _MAX_K), ow_full.T.reshape(B, S, _MAX_K),
            budget_full.reshape(B, S, 1), aux[0, 0])
